# baseline (speedup 1.0000x reference)
import functools

import jax
import jax.numpy as jnp
from jax import lax
from jax.experimental import pallas as pl
from jax.experimental.pallas import tpu as pltpu
from jax.experimental.pallas import tpu_sc as plsc

F32 = jnp.float32
I32 = jnp.int32

E = 320000
NHID = 128
NW = 32
EPW = E // NW
CH = 80
NCHUNK = EPW // CH

_MESH = plsc.VectorSubcoreMesh(core_axis_name="c", subcore_axis_name="s")


def _pad16(n):
    np_ = n + 80
    return np_ + (-np_ % 256)


def _lin_body(x_ref, w_ref, b_ref, fw_ref, h_ref, acc_ref):
    h = jnp.dot(x_ref[...], w_ref[...], preferred_element_type=F32) + b_ref[...]
    h_ref[...] = h
    acc_ref[...] = fw_ref[0, 0] * h


def _lin(x, W, b, fw0):
    n = x.shape[0]
    blk = 256
    grid = (n + blk - 1) // blk
    return pl.pallas_call(
        _lin_body,
        grid=(grid,),
        in_specs=[
            pl.BlockSpec((blk, NHID), lambda i: (i, 0)),
            pl.BlockSpec((NHID, NHID), lambda i: (0, 0)),
            pl.BlockSpec((1, NHID), lambda i: (0, 0)),
            pl.BlockSpec(memory_space=pltpu.SMEM),
        ],
        out_specs=[
            pl.BlockSpec((blk, NHID), lambda i: (i, 0)),
            pl.BlockSpec((blk, NHID), lambda i: (i, 0)),
        ],
        out_shape=[
            jax.ShapeDtypeStruct((n, NHID), F32),
            jax.ShapeDtypeStruct((n, NHID), F32),
        ],
    )(x, W, b.reshape(1, NHID), fw0)


def _axpy_body(acc_ref, h_ref, fw_ref, hout_ref, out_ref):
    h = h_ref[...]
    hout_ref[...] = h
    out_ref[...] = acc_ref[...] + fw_ref[0, 0] * h


def _axpy(acc, hp, fwi, n):
    blk = 256
    grid = (n + blk - 1) // blk
    return pl.pallas_call(
        _axpy_body,
        grid=(grid,),
        in_specs=[
            pl.BlockSpec((blk, NHID), lambda i: (i, 0)),
            pl.BlockSpec((blk, NHID), lambda i: (i, 0)),
            pl.BlockSpec(memory_space=pltpu.SMEM),
        ],
        out_specs=[
            pl.BlockSpec((blk, NHID), lambda i: (i, 0)),
            pl.BlockSpec((blk, NHID), lambda i: (i, 0)),
        ],
        out_shape=[
            jax.ShapeDtypeStruct((n, NHID), F32),
            jax.ShapeDtypeStruct((n, NHID), F32),
        ],
    )(acc, hp, fwi)


def _idcopy_body(x_ref, o_ref):
    o_ref[...] = x_ref[...]


def _idcopy(a):
    n = a.shape[0]
    rows = n // NHID
    a2 = a.reshape(rows, NHID)
    blk = 256
    out = pl.pallas_call(
        _idcopy_body,
        grid=((rows + blk - 1) // blk,),
        in_specs=[pl.BlockSpec((blk, NHID), lambda i: (i, 0))],
        out_specs=pl.BlockSpec((blk, NHID), lambda i: (i, 0)),
        out_shape=jax.ShapeDtypeStruct((rows, NHID), a.dtype),
    )(a2)
    return out.reshape(n)


def _score_body(x_ref, p_ref, s_ref):
    p = p_ref[...]
    nrm = jnp.sqrt(jnp.sum(p * p)) + 1e-12
    s_ref[...] = jnp.dot(x_ref[...], p, preferred_element_type=F32) / nrm


def _score(x, p):
    n = x.shape[0]
    blk = 512
    grid = (n + blk - 1) // blk
    s = pl.pallas_call(
        _score_body,
        grid=(grid,),
        in_specs=[
            pl.BlockSpec((blk, NHID), lambda i: (i, 0)),
            pl.BlockSpec((NHID, 1), lambda i: (0, 0)),
        ],
        out_specs=pl.BlockSpec((blk, 1), lambda i: (i, 0)),
        out_shape=jax.ShapeDtypeStruct((n, 1), F32),
    )(x, p.reshape(NHID, 1))
    return s.reshape(n)


def _pre1_body(src_ref, dst_ref, out_ref):
    lane = jax.lax.broadcasted_iota(I32, src_ref.shape, 1) % 64
    out_ref[...] = jnp.where(src_ref[...] != dst_ref[...], dst_ref[...],
                             10000 + lane)


def _pre1(src, dst):
    s2 = src.reshape(2500, NHID)
    d2 = dst.reshape(2500, NHID)
    blk = 256
    out = pl.pallas_call(
        _pre1_body,
        grid=((2500 + blk - 1) // blk,),
        in_specs=[
            pl.BlockSpec((blk, NHID), lambda i: (i, 0)),
            pl.BlockSpec((blk, NHID), lambda i: (i, 0)),
        ],
        out_specs=pl.BlockSpec((blk, NHID), lambda i: (i, 0)),
        out_shape=jax.ShapeDtypeStruct((2500, NHID), I32),
    )(s2, d2)
    return out.reshape(E)


def _mlp_body(x_ref, w1_ref, b1_ref, w2_ref, b2_ref, out_ref):
    sums = jnp.sum(x_ref[...], axis=0, keepdims=True)
    mean = sums / jnp.float32(x_ref.shape[0])
    h = jnp.dot(mean, w1_ref[...], preferred_element_type=F32) + b1_ref[...]
    h = jnp.maximum(h, 0.0)
    out_ref[...] = jnp.dot(h, w2_ref[...], preferred_element_type=F32) + b2_ref[...]


def _mlp(x, Wl1, bl1, Wl2, bl2):
    return pl.pallas_call(
        _mlp_body,
        out_shape=jax.ShapeDtypeStruct((1, 1), F32),
    )(x, Wl1, bl1.reshape(1, -1), Wl2, bl2.reshape(1, 1))


def _wid():
    return lax.axis_index("s") * 2 + lax.axis_index("c")


def _zero_rows(buf, nrows):
    z = jnp.zeros((16,), F32)

    def zrow(r, _):
        for j in range(NHID // 16):
            buf[r, pl.ds(j * 16, 16)] = z
        return 0
    lax.fori_loop(0, nrows, zrow, 0)


def _fill_copy(buf, dst_ref, start, count):
    nfull = count // CH
    rem = count - nfull * CH

    def cp(i, _):
        pltpu.sync_copy(buf.at[pl.ds(0, CH)],
                        dst_ref.at[pl.ds(start + i * CH, CH)])
        return 0
    lax.fori_loop(0, nfull, cp, 0)
    if rem:
        pltpu.sync_copy(buf.at[pl.ds(0, rem)],
                        dst_ref.at[pl.ds(start + nfull * CH, rem)])


PT = 12800
BLK = 3200
FLB = 1600


def _place_kernel(n, np_):
    PTOT = NW * PT

    def body(src_ref, dl_ref, addr_ref, pada_ref, rs_ref, rd_ref,
             sb, db, ab, vb, sem):
        w = _wid()
        iota = lax.iota(I32, 16)
        base = w * EPW

        def chunk(ci, _):
            off = base + ci * CH
            pltpu.sync_copy(src_ref.at[pl.ds(off, CH)], sb)
            pltpu.sync_copy(dl_ref.at[pl.ds(off, CH)], db)
            pltpu.sync_copy(addr_ref.at[pl.ds(off, CH)], ab)
            pltpu.sync_copy(sb, rs_ref.at[ab])
            pltpu.sync_copy(db, rd_ref.at[ab])
            return 0
        lax.fori_loop(0, NCHUNK, chunk, 0)

        pltpu.sync_copy(pada_ref.at[pl.ds(w * CH, CH)], ab)
        for t in range(CH // 16):
            vb[pl.ds(t * 16, 16)] = (iota + t * 16) % 64
        pltpu.sync_copy(vb, rs_ref.at[ab])
        rt = np_ // NW
        for t in range(CH // 16):
            vb[pl.ds(t * 16, 16)] = jnp.full((16,), rt, I32)
        pltpu.sync_copy(vb, rd_ref.at[ab])

    return functools.partial(
        pl.kernel, body,
        out_type=[
            jax.ShapeDtypeStruct((PTOT + 128,), I32),
            jax.ShapeDtypeStruct((PTOT + 128,), I32),
        ],
        mesh=_MESH,
        scratch_types=[
            pltpu.VMEM((CH,), I32),
            pltpu.VMEM((CH,), I32),
            pltpu.VMEM((CH,), I32),
            pltpu.VMEM((CH,), I32),
            pltpu.SemaphoreType.DMA,
        ],
    )()


def _accum2_kernel(n, np_):
    RT = np_ // NW
    RTA = RT + 8

    def body(h_ref, rs_ref, rd_ref, nch_ref, out_ref, sidx, dlb, rows, acc,
             cv, sem):
        w = _wid()

        _zero_rows(acc, RTA)

        pltpu.sync_copy(nch_ref.at[pl.ds(w * 16, 16)], cv)
        nch = cv[pl.ds(0, 16)][0]
        base = w * PT

        def chunk(ci, _):
            pltpu.sync_copy(rs_ref.at[pl.ds(base + ci * CH, CH)], sidx)
            pltpu.sync_copy(rd_ref.at[pl.ds(base + ci * CH, CH)], dlb)
            pltpu.async_copy(h_ref.at[sidx], rows, sem).wait()
            for j in range(CH // 16):
                dv = dlb[pl.ds(j * 16, 16)]
                for l in range(16):
                    r = dv[l]
                    e = j * 16 + l
                    for q in range(NHID // 16):
                        sl = pl.ds(q * 16, 16)
                        acc[r, sl] = acc[r, sl] + rows[e, sl]
            return 0
        lax.fori_loop(0, nch, chunk, 0)

        pltpu.sync_copy(acc.at[pl.ds(0, RT)], out_ref.at[pl.ds(w * RT, RT)])

    return functools.partial(
        pl.kernel, body,
        out_type=jax.ShapeDtypeStruct((np_, NHID), F32),
        mesh=_MESH,
        scratch_types=[
            pltpu.VMEM((CH,), I32),
            pltpu.VMEM((CH,), I32),
            pltpu.VMEM((CH, NHID), F32),
            pltpu.VMEM((RTA, NHID), F32),
            pltpu.VMEM((16,), I32),
            pltpu.SemaphoreType.DMA,
        ],
    )()


def _build_kernel(np_prev, ps):
    per_sub = ps // 16
    zcount = np_prev // 16

    def body(perm_ref, enc_ref, idxb, valb, sem):
        c = lax.axis_index("c")
        s = lax.axis_index("s")
        iota = lax.iota(I32, 16)

        @pl.when(c == 0)
        def _():
            neg = jnp.full((16,), -1, I32)
            for j in range(CH // 16):
                valb[pl.ds(j * 16, 16)] = neg
            _fill_copy(valb, enc_ref, s * zcount, zcount)
            plsc.subcore_barrier()

            nchunk = per_sub // CH

            def sc(ci, _):
                base = s * per_sub + ci * CH
                pltpu.sync_copy(perm_ref.at[pl.ds(base, CH)], idxb)
                for j in range(CH // 16):
                    valb[pl.ds(j * 16, 16)] = base + j * 16 + iota
                pltpu.sync_copy(valb, enc_ref.at[idxb])
                return 0
            lax.fori_loop(0, nchunk, sc, 0)

    return functools.partial(
        pl.kernel, body,
        out_type=jax.ShapeDtypeStruct((np_prev,), I32),
        mesh=_MESH,
        scratch_types=[
            pltpu.VMEM((CH,), I32),
            pltpu.VMEM((CH,), I32),
            pltpu.SemaphoreType.DMA,
        ],
    )()


def _preprocess_kernel(n_new):
    def body(src_ref, dst_ref, enc_ref, srcn_ref, dstn_ref,
             sb, db, ns, nd, so, do, sem, sem2):
        w = _wid()
        iota = lax.iota(I32, 16)
        base = w * EPW

        def chunk(ci, _):
            off = base + ci * CH
            pltpu.sync_copy(src_ref.at[pl.ds(off, CH)], sb)
            pltpu.sync_copy(dst_ref.at[pl.ds(off, CH)], db)
            cp1 = pltpu.async_copy(enc_ref.at[sb], ns, sem)
            cp2 = pltpu.async_copy(enc_ref.at[db], nd, sem2)
            cp1.wait()
            cp2.wait()
            for j in range(CH // 16):
                sl = pl.ds(j * 16, 16)
                nsv = ns[sl]
                ndv = nd[sl]
                valid = (nsv >= 0) & (ndv >= 0)
                spread = (j % 4) * 16 + iota
                so[sl] = jnp.where(valid, nsv, spread)
                do[sl] = jnp.where(valid, ndv, n_new + spread)
            pltpu.sync_copy(so, srcn_ref.at[pl.ds(off, CH)])
            pltpu.sync_copy(do, dstn_ref.at[pl.ds(off, CH)])
            return 0
        lax.fori_loop(0, NCHUNK, chunk, 0)

    return functools.partial(
        pl.kernel, body,
        out_type=[
            jax.ShapeDtypeStruct((E,), I32),
            jax.ShapeDtypeStruct((E,), I32),
        ],
        mesh=_MESH,
        scratch_types=[pltpu.VMEM((CH,), I32) for _ in range(6)]
        + [pltpu.SemaphoreType.DMA, pltpu.SemaphoreType.DMA],
    )()


def _gate_kernel(n, k, cg, ps):
    per_w = ps // NW

    def body(x_ref, perm_ref, out_ref, idxb, rows, sem):
        w = _wid()
        nchunk = per_w // cg

        def chunk(ci, _):
            base = w * per_w + ci * cg
            pltpu.sync_copy(perm_ref.at[pl.ds(base, cg)], idxb)
            pltpu.async_copy(x_ref.at[idxb], rows, sem).wait()
            pltpu.sync_copy(rows, out_ref.at[pl.ds(base, cg)])
            return 0
        lax.fori_loop(0, nchunk, chunk, 0)

    return functools.partial(
        pl.kernel, body,
        out_type=jax.ShapeDtypeStruct((ps, NHID), F32),
        mesh=_MESH,
        scratch_types=[
            pltpu.VMEM((cg,), I32),
            pltpu.VMEM((cg, NHID), F32),
            pltpu.SemaphoreType.DMA,
        ],
    )()


def _gatemul_body(x_ref, g_ref, out_ref):
    out_ref[...] = x_ref[...] * g_ref[...]


def _gatemul(xg, gate, k):
    blk = 256
    grid = (k + blk - 1) // blk
    return pl.pallas_call(
        _gatemul_body,
        grid=(grid,),
        in_specs=[
            pl.BlockSpec((blk, NHID), lambda i: (i, 0)),
            pl.BlockSpec((blk, 1), lambda i: (i, 0)),
        ],
        out_specs=pl.BlockSpec((blk, NHID), lambda i: (i, 0)),
        out_shape=jax.ShapeDtypeStruct((k, NHID), F32),
    )(xg, gate.reshape(k, 1))


def _conv(h0, acc, srcd, dstd, fw, n, np_):
    RT = np_ // NW
    PTOT = NW * PT
    valid = dstd < n
    owner = jnp.where(valid, dstd, 0) // RT
    oh = ((owner[:, None] == jnp.arange(NW, dtype=I32)[None, :])
          & valid[:, None]).astype(I32)
    ranks = jnp.cumsum(oh, axis=0)
    pos = jnp.sum(oh * ranks, axis=1) - 1
    counts = ranks[-1]
    addr = jnp.where(valid, owner * PT + pos,
                     PTOT + (jnp.arange(E, dtype=I32) % 64))
    dl = jnp.where(valid, dstd - owner * RT, 0)
    nch = (jnp.minimum(counts, PT - CH) + (CH - 1)) // CH
    nchv = jnp.broadcast_to(nch[:, None], (NW, 16)).reshape(NW * 16)
    pada = (jnp.arange(NW, dtype=I32)[:, None] * PT
            + jnp.minimum(counts, PT - CH)[:, None]
            + jnp.arange(CH, dtype=I32)[None, :]).reshape(NW * CH)

    routed_s, routed_d = _place_kernel(n, np_)(srcd, dl, addr, pada)
    accum = _accum2_kernel(n, np_)
    h = h0
    for i in range(1, int(fw.shape[0])):
        hp = accum(h, routed_s, routed_d, nchv)
        h, acc = _axpy(acc, hp[:n], fw[i].reshape(1, 1), n)
    return acc


def _gate_pads(k, cg):
    span32 = (k + NW - 1) // NW
    per_w = (span32 + cg - 1) // cg * cg
    return per_w * NW


def _build_pads(k):
    span16 = (k + 15) // 16
    per_sub = (span16 + CH - 1) // CH * CH
    return per_sub * 16


def _pool(x, srcd, dstd, p, n, np_, k, cg, last):
    score = _score(x, p)
    topv, perm = lax.top_k(score, k)
    gate = jnp.tanh(topv)

    psg = _gate_pads(k, cg)
    pad_g = psg - k
    perm_g = jnp.concatenate(
        [perm, (jnp.arange(pad_g, dtype=I32) % 64)])
    xg = _gate_kernel(n, k, cg, psg)(x, perm_g)[:k]
    xk = _gatemul(xg, gate, k)

    if last:
        return xk, None, None, perm

    psb = _build_pads(k)
    pad_b = psb - k
    free = np_ - (n + 64)
    perm_b = jnp.concatenate(
        [perm, n + 64 + (jnp.arange(pad_b, dtype=I32) % free)])
    enc = _build_kernel(np_, psb)(perm_b)
    srcn, dstn = _preprocess_kernel(k)(srcd, dstd, enc)
    return xk, srcn, dstn, perm


def kernel(x, edge_index, edge_attr, batch, W_lump, b_lump, W1, b1, fw1, p1,
           W2, b2, fw2, p2, W3, b3, fw3, p3, Wl1, bl1, Wl2, bl2):
    del edge_attr, batch
    src = edge_index[0]
    dst = edge_index[1]

    N1, N2, N3, N4 = 10000, 5000, 2500, 1250
    NP1, NP2, NP3 = _pad16(N1), _pad16(N2), _pad16(N3)

    dst_e1 = _pre1(src, dst)
    src_e1 = src

    x0, _ = _lin(x, W_lump, b_lump, jnp.zeros((1, 1), F32))
    h0, acc = _lin(x0, W1, b1, fw1[0].reshape(1, 1))
    x1 = _conv(h0, acc, src_e1, dst_e1, fw1, N1, NP1)
    xk1, src_e2, dst_e2, perm1 = _pool(x1, src_e1, dst_e1, p1, N1, NP1, N2,
                                       CH, False)

    h0, acc = _lin(xk1, W2, b2, fw2[0].reshape(1, 1))
    x2 = _conv(h0, acc, src_e2, dst_e2, fw2, N2, NP2)
    xk2, src_e3, dst_e3, perm2 = _pool(x2, src_e2, dst_e2, p2, N2, NP2, N3,
                                       CH, False)

    h0, acc = _lin(xk2, W3, b3, fw3[0].reshape(1, 1))
    x3 = _conv(h0, acc, src_e3, dst_e3, fw3, N3, NP3)
    xk3, _, _, perm3 = _pool(x3, None, None, p3, N3, NP3, N4, 40, True)

    out = _mlp(xk3, Wl1, bl1, Wl2, bl2)
    return (out, perm1, perm2, perm3)


# speedup vs baseline: 1.0145x; 1.0145x over previous
import functools

import jax
import jax.numpy as jnp
from jax import lax
from jax.experimental import pallas as pl
from jax.experimental.pallas import tpu as pltpu
from jax.experimental.pallas import tpu_sc as plsc

F32 = jnp.float32
I32 = jnp.int32

E = 320000
NHID = 128
NW = 32
EPW = E // NW
CH = 80
NCHUNK = EPW // CH

_MESH = plsc.VectorSubcoreMesh(core_axis_name="c", subcore_axis_name="s")


def _pad16(n):
    np_ = n + 80
    return np_ + (-np_ % 256)


def _lin_body(x_ref, w_ref, b_ref, fw_ref, h_ref, acc_ref):
    h = jnp.dot(x_ref[...], w_ref[...], preferred_element_type=F32) + b_ref[...]
    h_ref[...] = h
    acc_ref[...] = fw_ref[0, 0] * h


def _lin(x, W, b, fw0):
    n = x.shape[0]
    blk = 256
    grid = (n + blk - 1) // blk
    return pl.pallas_call(
        _lin_body,
        grid=(grid,),
        in_specs=[
            pl.BlockSpec((blk, NHID), lambda i: (i, 0)),
            pl.BlockSpec((NHID, NHID), lambda i: (0, 0)),
            pl.BlockSpec((1, NHID), lambda i: (0, 0)),
            pl.BlockSpec(memory_space=pltpu.SMEM),
        ],
        out_specs=[
            pl.BlockSpec((blk, NHID), lambda i: (i, 0)),
            pl.BlockSpec((blk, NHID), lambda i: (i, 0)),
        ],
        out_shape=[
            jax.ShapeDtypeStruct((n, NHID), F32),
            jax.ShapeDtypeStruct((n, NHID), F32),
        ],
    )(x, W, b.reshape(1, NHID), fw0)


def _axpy_body(acc_ref, h_ref, fw_ref, hout_ref, out_ref):
    h = h_ref[...]
    hout_ref[...] = h
    out_ref[...] = acc_ref[...] + fw_ref[0, 0] * h


def _axpy(acc, hp, fwi, n):
    blk = 256
    grid = (n + blk - 1) // blk
    return pl.pallas_call(
        _axpy_body,
        grid=(grid,),
        in_specs=[
            pl.BlockSpec((blk, NHID), lambda i: (i, 0)),
            pl.BlockSpec((blk, NHID), lambda i: (i, 0)),
            pl.BlockSpec(memory_space=pltpu.SMEM),
        ],
        out_specs=[
            pl.BlockSpec((blk, NHID), lambda i: (i, 0)),
            pl.BlockSpec((blk, NHID), lambda i: (i, 0)),
        ],
        out_shape=[
            jax.ShapeDtypeStruct((n, NHID), F32),
            jax.ShapeDtypeStruct((n, NHID), F32),
        ],
    )(acc, hp, fwi)


def _idcopy_body(x_ref, o_ref):
    o_ref[...] = x_ref[...]


def _idcopy(a):
    n = a.shape[0]
    rows = n // NHID
    a2 = a.reshape(rows, NHID)
    blk = 256
    out = pl.pallas_call(
        _idcopy_body,
        grid=((rows + blk - 1) // blk,),
        in_specs=[pl.BlockSpec((blk, NHID), lambda i: (i, 0))],
        out_specs=pl.BlockSpec((blk, NHID), lambda i: (i, 0)),
        out_shape=jax.ShapeDtypeStruct((rows, NHID), a.dtype),
    )(a2)
    return out.reshape(n)


def _score_body(x_ref, p_ref, s_ref):
    p = p_ref[...]
    nrm = jnp.sqrt(jnp.sum(p * p)) + 1e-12
    s_ref[...] = jnp.dot(x_ref[...], p, preferred_element_type=F32) / nrm


def _score(x, p):
    n = x.shape[0]
    blk = 512
    grid = (n + blk - 1) // blk
    s = pl.pallas_call(
        _score_body,
        grid=(grid,),
        in_specs=[
            pl.BlockSpec((blk, NHID), lambda i: (i, 0)),
            pl.BlockSpec((NHID, 1), lambda i: (0, 0)),
        ],
        out_specs=pl.BlockSpec((blk, 1), lambda i: (i, 0)),
        out_shape=jax.ShapeDtypeStruct((n, 1), F32),
    )(x, p.reshape(NHID, 1))
    return s.reshape(n)


def _pre1_body(src_ref, dst_ref, out_ref):
    lane = jax.lax.broadcasted_iota(I32, src_ref.shape, 1) % 64
    out_ref[...] = jnp.where(src_ref[...] != dst_ref[...], dst_ref[...],
                             10000 + lane)


def _pre1(src, dst):
    s2 = src.reshape(2500, NHID)
    d2 = dst.reshape(2500, NHID)
    blk = 256
    out = pl.pallas_call(
        _pre1_body,
        grid=((2500 + blk - 1) // blk,),
        in_specs=[
            pl.BlockSpec((blk, NHID), lambda i: (i, 0)),
            pl.BlockSpec((blk, NHID), lambda i: (i, 0)),
        ],
        out_specs=pl.BlockSpec((blk, NHID), lambda i: (i, 0)),
        out_shape=jax.ShapeDtypeStruct((2500, NHID), I32),
    )(s2, d2)
    return out.reshape(E)


def _mlp_body(x_ref, w1_ref, b1_ref, w2_ref, b2_ref, out_ref):
    sums = jnp.sum(x_ref[...], axis=0, keepdims=True)
    mean = sums / jnp.float32(x_ref.shape[0])
    h = jnp.dot(mean, w1_ref[...], preferred_element_type=F32) + b1_ref[...]
    h = jnp.maximum(h, 0.0)
    out_ref[...] = jnp.dot(h, w2_ref[...], preferred_element_type=F32) + b2_ref[...]


def _mlp(x, Wl1, bl1, Wl2, bl2):
    return pl.pallas_call(
        _mlp_body,
        out_shape=jax.ShapeDtypeStruct((1, 1), F32),
    )(x, Wl1, bl1.reshape(1, -1), Wl2, bl2.reshape(1, 1))


def _wid():
    return lax.axis_index("s") * 2 + lax.axis_index("c")


def _zero_rows(buf, nrows):
    z = jnp.zeros((16,), F32)

    def zrow(r, _):
        for j in range(NHID // 16):
            buf[r, pl.ds(j * 16, 16)] = z
        return 0
    lax.fori_loop(0, nrows, zrow, 0)


def _fill_copy(buf, dst_ref, start, count):
    nfull = count // CH
    rem = count - nfull * CH

    def cp(i, _):
        pltpu.sync_copy(buf.at[pl.ds(0, CH)],
                        dst_ref.at[pl.ds(start + i * CH, CH)])
        return 0
    lax.fori_loop(0, nfull, cp, 0)
    if rem:
        pltpu.sync_copy(buf.at[pl.ds(0, rem)],
                        dst_ref.at[pl.ds(start + nfull * CH, rem)])


PT = 12800
BLK = 3200
FLB = 1600


def _place_kernel(n, np_):
    PTOT = NW * PT

    def body(src_ref, dl_ref, addr_ref, pada_ref, rs_ref, rd_ref,
             sb, db, ab, vb, sem):
        w = _wid()
        iota = lax.iota(I32, 16)
        base = w * EPW

        def chunk(ci, _):
            off = base + ci * CH
            pltpu.sync_copy(src_ref.at[pl.ds(off, CH)], sb)
            pltpu.sync_copy(dl_ref.at[pl.ds(off, CH)], db)
            pltpu.sync_copy(addr_ref.at[pl.ds(off, CH)], ab)
            pltpu.sync_copy(sb, rs_ref.at[ab])
            pltpu.sync_copy(db, rd_ref.at[ab])
            return 0
        lax.fori_loop(0, NCHUNK, chunk, 0)

        pltpu.sync_copy(pada_ref.at[pl.ds(w * CH, CH)], ab)
        for t in range(CH // 16):
            vb[pl.ds(t * 16, 16)] = (iota + t * 16) % 64
        pltpu.sync_copy(vb, rs_ref.at[ab])
        rt = np_ // NW
        for t in range(CH // 16):
            vb[pl.ds(t * 16, 16)] = jnp.full((16,), rt, I32)
        pltpu.sync_copy(vb, rd_ref.at[ab])

    return functools.partial(
        pl.kernel, body,
        out_type=[
            jax.ShapeDtypeStruct((PTOT + 128,), I32),
            jax.ShapeDtypeStruct((PTOT + 128,), I32),
        ],
        mesh=_MESH,
        scratch_types=[
            pltpu.VMEM((CH,), I32),
            pltpu.VMEM((CH,), I32),
            pltpu.VMEM((CH,), I32),
            pltpu.VMEM((CH,), I32),
            pltpu.SemaphoreType.DMA,
        ],
    )()


def _accum2_kernel(n, np_):
    RT = np_ // NW
    RTA = RT + 8

    def body(h_ref, rs_ref, rd_ref, nch_ref, out_ref, sidx, dlb, rows, acc,
             cv, sem):
        w = _wid()

        _zero_rows(acc, RTA)

        pltpu.sync_copy(nch_ref.at[pl.ds(w * 16, 16)], cv)
        nch = cv[pl.ds(0, 16)][0]
        base = w * PT

        def chunk(ci, _):
            pltpu.sync_copy(rs_ref.at[pl.ds(base + ci * CH, CH)], sidx)
            pltpu.sync_copy(rd_ref.at[pl.ds(base + ci * CH, CH)], dlb)
            pltpu.async_copy(h_ref.at[sidx], rows, sem).wait()
            for j in range(CH // 16):
                dv = dlb[pl.ds(j * 16, 16)]
                for l in range(16):
                    r = dv[l]
                    e = j * 16 + l
                    for q in range(NHID // 16):
                        sl = pl.ds(q * 16, 16)
                        acc[r, sl] = acc[r, sl] + rows[e, sl]
            return 0
        lax.fori_loop(0, nch, chunk, 0)

        pltpu.sync_copy(acc.at[pl.ds(0, RT)], out_ref.at[pl.ds(w * RT, RT)])

    return functools.partial(
        pl.kernel, body,
        out_type=jax.ShapeDtypeStruct((np_, NHID), F32),
        mesh=_MESH,
        scratch_types=[
            pltpu.VMEM((CH,), I32),
            pltpu.VMEM((CH,), I32),
            pltpu.VMEM((CH, NHID), F32),
            pltpu.VMEM((RTA, NHID), F32),
            pltpu.VMEM((16,), I32),
            pltpu.SemaphoreType.DMA,
        ],
    )()


def _build_kernel(np_prev, ps):
    per_sub = ps // 16
    zcount = np_prev // 16

    def body(perm_ref, enc_ref, idxb, valb, sem):
        c = lax.axis_index("c")
        s = lax.axis_index("s")
        iota = lax.iota(I32, 16)

        @pl.when(c == 0)
        def _():
            neg = jnp.full((16,), -1, I32)
            for j in range(CH // 16):
                valb[pl.ds(j * 16, 16)] = neg
            _fill_copy(valb, enc_ref, s * zcount, zcount)
            plsc.subcore_barrier()

            nchunk = per_sub // CH

            def sc(ci, _):
                base = s * per_sub + ci * CH
                pltpu.sync_copy(perm_ref.at[pl.ds(base, CH)], idxb)
                for j in range(CH // 16):
                    valb[pl.ds(j * 16, 16)] = base + j * 16 + iota
                pltpu.sync_copy(valb, enc_ref.at[idxb])
                return 0
            lax.fori_loop(0, nchunk, sc, 0)

    return functools.partial(
        pl.kernel, body,
        out_type=jax.ShapeDtypeStruct((np_prev,), I32),
        mesh=_MESH,
        scratch_types=[
            pltpu.VMEM((CH,), I32),
            pltpu.VMEM((CH,), I32),
            pltpu.SemaphoreType.DMA,
        ],
    )()


def _preprocess_kernel(n_new):
    def body(src_ref, dst_ref, enc_ref, srcn_ref, dstn_ref,
             sb, db, ns, nd, so, do, sem, sem2):
        w = _wid()
        iota = lax.iota(I32, 16)
        base = w * EPW

        def chunk(ci, _):
            off = base + ci * CH
            pltpu.sync_copy(src_ref.at[pl.ds(off, CH)], sb)
            pltpu.sync_copy(dst_ref.at[pl.ds(off, CH)], db)
            cp1 = pltpu.async_copy(enc_ref.at[sb], ns, sem)
            cp2 = pltpu.async_copy(enc_ref.at[db], nd, sem2)
            cp1.wait()
            cp2.wait()
            for j in range(CH // 16):
                sl = pl.ds(j * 16, 16)
                nsv = ns[sl]
                ndv = nd[sl]
                valid = (nsv >= 0) & (ndv >= 0)
                spread = (j % 4) * 16 + iota
                so[sl] = jnp.where(valid, nsv, spread)
                do[sl] = jnp.where(valid, ndv, n_new + spread)
            pltpu.sync_copy(so, srcn_ref.at[pl.ds(off, CH)])
            pltpu.sync_copy(do, dstn_ref.at[pl.ds(off, CH)])
            return 0
        lax.fori_loop(0, NCHUNK, chunk, 0)

    return functools.partial(
        pl.kernel, body,
        out_type=[
            jax.ShapeDtypeStruct((E,), I32),
            jax.ShapeDtypeStruct((E,), I32),
        ],
        mesh=_MESH,
        scratch_types=[pltpu.VMEM((CH,), I32) for _ in range(6)]
        + [pltpu.SemaphoreType.DMA, pltpu.SemaphoreType.DMA],
    )()


def _gate_kernel(n, k, cg, ps):
    per_w = ps // NW

    def body(x_ref, perm_ref, out_ref, idxb, rows, sem):
        w = _wid()
        nchunk = per_w // cg

        def chunk(ci, _):
            base = w * per_w + ci * cg
            pltpu.sync_copy(perm_ref.at[pl.ds(base, cg)], idxb)
            pltpu.async_copy(x_ref.at[idxb], rows, sem).wait()
            pltpu.sync_copy(rows, out_ref.at[pl.ds(base, cg)])
            return 0
        lax.fori_loop(0, nchunk, chunk, 0)

    return functools.partial(
        pl.kernel, body,
        out_type=jax.ShapeDtypeStruct((ps, NHID), F32),
        mesh=_MESH,
        scratch_types=[
            pltpu.VMEM((cg,), I32),
            pltpu.VMEM((cg, NHID), F32),
            pltpu.SemaphoreType.DMA,
        ],
    )()


def _gatemul_body(x_ref, g_ref, out_ref):
    out_ref[...] = x_ref[...] * g_ref[...]


def _gatemul(xg, gate, k):
    blk = 256
    grid = (k + blk - 1) // blk
    return pl.pallas_call(
        _gatemul_body,
        grid=(grid,),
        in_specs=[
            pl.BlockSpec((blk, NHID), lambda i: (i, 0)),
            pl.BlockSpec((blk, 1), lambda i: (i, 0)),
        ],
        out_specs=pl.BlockSpec((blk, NHID), lambda i: (i, 0)),
        out_shape=jax.ShapeDtypeStruct((k, NHID), F32),
    )(xg, gate.reshape(k, 1))


def _conv(h0, acc, srcd, dstd, fw, n, np_):
    RT = np_ // NW
    PTOT = NW * PT
    valid = dstd < n
    owner = jnp.where(valid, dstd, 0) // RT
    oh = ((owner[:, None] == jnp.arange(NW, dtype=I32)[None, :])
          & valid[:, None]).astype(I32).reshape(E // 128, 128, NW)
    within = jnp.cumsum(oh, axis=1)
    blk_counts = within[:, -1, :]
    base_blocks = jnp.cumsum(blk_counts, axis=0) - blk_counts
    pos3 = base_blocks[:, None, :] + within
    pos = jnp.sum(pos3 * oh, axis=2).reshape(E) - 1
    counts = blk_counts.sum(axis=0)
    addr = jnp.where(valid, owner * PT + pos,
                     PTOT + (jnp.arange(E, dtype=I32) % 64))
    dl = jnp.where(valid, dstd - owner * RT, 0)
    nch = (jnp.minimum(counts, PT - CH) + (CH - 1)) // CH
    nchv = jnp.broadcast_to(nch[:, None], (NW, 16)).reshape(NW * 16)
    pada = (jnp.arange(NW, dtype=I32)[:, None] * PT
            + jnp.minimum(counts, PT - CH)[:, None]
            + jnp.arange(CH, dtype=I32)[None, :]).reshape(NW * CH)

    routed_s, routed_d = _place_kernel(n, np_)(srcd, dl, addr, pada)
    accum = _accum2_kernel(n, np_)
    h = h0
    for i in range(1, int(fw.shape[0])):
        hp = accum(h, routed_s, routed_d, nchv)
        h, acc = _axpy(acc, hp[:n], fw[i].reshape(1, 1), n)
    return acc


def _gate_pads(k, cg):
    span32 = (k + NW - 1) // NW
    per_w = (span32 + cg - 1) // cg * cg
    return per_w * NW


def _build_pads(k):
    span16 = (k + 15) // 16
    per_sub = (span16 + CH - 1) // CH * CH
    return per_sub * 16


def _pool(x, srcd, dstd, p, n, np_, k, cg, last):
    score = _score(x, p)
    topv, perm = lax.top_k(score, k)
    gate = jnp.tanh(topv)

    psg = _gate_pads(k, cg)
    pad_g = psg - k
    perm_g = jnp.concatenate(
        [perm, (jnp.arange(pad_g, dtype=I32) % 64)])
    xg = _gate_kernel(n, k, cg, psg)(x, perm_g)[:k]
    xk = _gatemul(xg, gate, k)

    if last:
        return xk, None, None, perm

    psb = _build_pads(k)
    pad_b = psb - k
    free = np_ - (n + 64)
    perm_b = jnp.concatenate(
        [perm, n + 64 + (jnp.arange(pad_b, dtype=I32) % free)])
    enc = _build_kernel(np_, psb)(perm_b)
    srcn, dstn = _preprocess_kernel(k)(srcd, dstd, enc)
    return xk, srcn, dstn, perm


def kernel(x, edge_index, edge_attr, batch, W_lump, b_lump, W1, b1, fw1, p1,
           W2, b2, fw2, p2, W3, b3, fw3, p3, Wl1, bl1, Wl2, bl2):
    del edge_attr, batch
    src = edge_index[0]
    dst = edge_index[1]

    N1, N2, N3, N4 = 10000, 5000, 2500, 1250
    NP1, NP2, NP3 = _pad16(N1), _pad16(N2), _pad16(N3)

    dst_e1 = _pre1(src, dst)
    src_e1 = src

    x0, _ = _lin(x, W_lump, b_lump, jnp.zeros((1, 1), F32))
    h0, acc = _lin(x0, W1, b1, fw1[0].reshape(1, 1))
    x1 = _conv(h0, acc, src_e1, dst_e1, fw1, N1, NP1)
    xk1, src_e2, dst_e2, perm1 = _pool(x1, src_e1, dst_e1, p1, N1, NP1, N2,
                                       CH, False)

    h0, acc = _lin(xk1, W2, b2, fw2[0].reshape(1, 1))
    x2 = _conv(h0, acc, src_e2, dst_e2, fw2, N2, NP2)
    xk2, src_e3, dst_e3, perm2 = _pool(x2, src_e2, dst_e2, p2, N2, NP2, N3,
                                       CH, False)

    h0, acc = _lin(xk2, W3, b3, fw3[0].reshape(1, 1))
    x3 = _conv(h0, acc, src_e3, dst_e3, fw3, N3, NP3)
    xk3, _, _, perm3 = _pool(x3, None, None, p3, N3, NP3, N4, 40, True)

    out = _mlp(xk3, Wl1, bl1, Wl2, bl2)
    return (out, perm1, perm2, perm3)


# speedup vs baseline: 1.0220x; 1.0074x over previous
import functools

import jax
import jax.numpy as jnp
from jax import lax
from jax.experimental import pallas as pl
from jax.experimental.pallas import tpu as pltpu
from jax.experimental.pallas import tpu_sc as plsc

F32 = jnp.float32
I32 = jnp.int32

E = 320000
NHID = 128
NW = 32
EPW = E // NW
CH = 80
NCHUNK = EPW // CH

_MESH = plsc.VectorSubcoreMesh(core_axis_name="c", subcore_axis_name="s")


def _pad16(n):
    np_ = n + 80
    return np_ + (-np_ % 256)


def _lin_body(x_ref, w_ref, b_ref, fw_ref, h_ref, acc_ref):
    h = jnp.dot(x_ref[...], w_ref[...], preferred_element_type=F32) + b_ref[...]
    h_ref[...] = h
    acc_ref[...] = fw_ref[0, 0] * h


def _lin(x, W, b, fw0):
    n = x.shape[0]
    blk = 256
    grid = (n + blk - 1) // blk
    return pl.pallas_call(
        _lin_body,
        grid=(grid,),
        in_specs=[
            pl.BlockSpec((blk, NHID), lambda i: (i, 0)),
            pl.BlockSpec((NHID, NHID), lambda i: (0, 0)),
            pl.BlockSpec((1, NHID), lambda i: (0, 0)),
            pl.BlockSpec(memory_space=pltpu.SMEM),
        ],
        out_specs=[
            pl.BlockSpec((blk, NHID), lambda i: (i, 0)),
            pl.BlockSpec((blk, NHID), lambda i: (i, 0)),
        ],
        out_shape=[
            jax.ShapeDtypeStruct((n, NHID), F32),
            jax.ShapeDtypeStruct((n, NHID), F32),
        ],
    )(x, W, b.reshape(1, NHID), fw0)


def _axpy_body(acc_ref, h_ref, fw_ref, hout_ref, out_ref):
    h = h_ref[...]
    hout_ref[...] = h
    out_ref[...] = acc_ref[...] + fw_ref[0, 0] * h


def _axpy(acc, hp, fwi, n):
    blk = 256
    grid = (n + blk - 1) // blk
    return pl.pallas_call(
        _axpy_body,
        grid=(grid,),
        in_specs=[
            pl.BlockSpec((blk, NHID), lambda i: (i, 0)),
            pl.BlockSpec((blk, NHID), lambda i: (i, 0)),
            pl.BlockSpec(memory_space=pltpu.SMEM),
        ],
        out_specs=[
            pl.BlockSpec((blk, NHID), lambda i: (i, 0)),
            pl.BlockSpec((blk, NHID), lambda i: (i, 0)),
        ],
        out_shape=[
            jax.ShapeDtypeStruct((n, NHID), F32),
            jax.ShapeDtypeStruct((n, NHID), F32),
        ],
    )(acc, hp, fwi)


def _idcopy_body(x_ref, o_ref):
    o_ref[...] = x_ref[...]


def _idcopy(a):
    n = a.shape[0]
    rows = n // NHID
    a2 = a.reshape(rows, NHID)
    blk = 256
    out = pl.pallas_call(
        _idcopy_body,
        grid=((rows + blk - 1) // blk,),
        in_specs=[pl.BlockSpec((blk, NHID), lambda i: (i, 0))],
        out_specs=pl.BlockSpec((blk, NHID), lambda i: (i, 0)),
        out_shape=jax.ShapeDtypeStruct((rows, NHID), a.dtype),
    )(a2)
    return out.reshape(n)


def _score_body(x_ref, p_ref, s_ref):
    p = p_ref[...]
    nrm = jnp.sqrt(jnp.sum(p * p)) + 1e-12
    s_ref[...] = jnp.dot(x_ref[...], p, preferred_element_type=F32) / nrm


def _score(x, p):
    n = x.shape[0]
    blk = 512
    grid = (n + blk - 1) // blk
    s = pl.pallas_call(
        _score_body,
        grid=(grid,),
        in_specs=[
            pl.BlockSpec((blk, NHID), lambda i: (i, 0)),
            pl.BlockSpec((NHID, 1), lambda i: (0, 0)),
        ],
        out_specs=pl.BlockSpec((blk, 1), lambda i: (i, 0)),
        out_shape=jax.ShapeDtypeStruct((n, 1), F32),
    )(x, p.reshape(NHID, 1))
    return s.reshape(n)


def _pre1_body(src_ref, dst_ref, out_ref):
    lane = jax.lax.broadcasted_iota(I32, src_ref.shape, 1) % 64
    out_ref[...] = jnp.where(src_ref[...] != dst_ref[...], dst_ref[...],
                             10000 + lane)


def _pre1(src, dst):
    s2 = src.reshape(2500, NHID)
    d2 = dst.reshape(2500, NHID)
    blk = 256
    out = pl.pallas_call(
        _pre1_body,
        grid=((2500 + blk - 1) // blk,),
        in_specs=[
            pl.BlockSpec((blk, NHID), lambda i: (i, 0)),
            pl.BlockSpec((blk, NHID), lambda i: (i, 0)),
        ],
        out_specs=pl.BlockSpec((blk, NHID), lambda i: (i, 0)),
        out_shape=jax.ShapeDtypeStruct((2500, NHID), I32),
    )(s2, d2)
    return out.reshape(E)


def _mlp_body(x_ref, w1_ref, b1_ref, w2_ref, b2_ref, out_ref):
    sums = jnp.sum(x_ref[...], axis=0, keepdims=True)
    mean = sums / jnp.float32(x_ref.shape[0])
    h = jnp.dot(mean, w1_ref[...], preferred_element_type=F32) + b1_ref[...]
    h = jnp.maximum(h, 0.0)
    out_ref[...] = jnp.dot(h, w2_ref[...], preferred_element_type=F32) + b2_ref[...]


def _mlp(x, Wl1, bl1, Wl2, bl2):
    return pl.pallas_call(
        _mlp_body,
        out_shape=jax.ShapeDtypeStruct((1, 1), F32),
    )(x, Wl1, bl1.reshape(1, -1), Wl2, bl2.reshape(1, 1))


def _wid():
    return lax.axis_index("s") * 2 + lax.axis_index("c")


def _zero_rows(buf, nrows):
    z = jnp.zeros((16,), F32)

    def zrow(r, _):
        for j in range(NHID // 16):
            buf[r, pl.ds(j * 16, 16)] = z
        return 0
    lax.fori_loop(0, nrows, zrow, 0)


def _fill_copy(buf, dst_ref, start, count):
    nfull = count // CH
    rem = count - nfull * CH

    def cp(i, _):
        pltpu.sync_copy(buf.at[pl.ds(0, CH)],
                        dst_ref.at[pl.ds(start + i * CH, CH)])
        return 0
    lax.fori_loop(0, nfull, cp, 0)
    if rem:
        pltpu.sync_copy(buf.at[pl.ds(0, rem)],
                        dst_ref.at[pl.ds(start + nfull * CH, rem)])


PT = 12800
BLK = 3200
FLB = 1600


def _place_kernel(n, np_):
    PTOT = NW * PT

    def body(src_ref, dl_ref, addr_ref, pada_ref, rs_ref, rd_ref,
             sb, db, ab, vb, sem):
        w = _wid()
        iota = lax.iota(I32, 16)
        base = w * EPW

        def chunk(ci, _):
            off = base + ci * CH
            pltpu.sync_copy(src_ref.at[pl.ds(off, CH)], sb)
            pltpu.sync_copy(dl_ref.at[pl.ds(off, CH)], db)
            pltpu.sync_copy(addr_ref.at[pl.ds(off, CH)], ab)
            pltpu.sync_copy(sb, rs_ref.at[ab])
            pltpu.sync_copy(db, rd_ref.at[ab])
            return 0
        lax.fori_loop(0, NCHUNK, chunk, 0)

        pltpu.sync_copy(pada_ref.at[pl.ds(w * CH, CH)], ab)
        for t in range(CH // 16):
            vb[pl.ds(t * 16, 16)] = (iota + t * 16) % 64
        pltpu.sync_copy(vb, rs_ref.at[ab])
        rt = np_ // NW
        for t in range(CH // 16):
            vb[pl.ds(t * 16, 16)] = jnp.full((16,), rt, I32)
        pltpu.sync_copy(vb, rd_ref.at[ab])

    return functools.partial(
        pl.kernel, body,
        out_type=[
            jax.ShapeDtypeStruct((PTOT + 128,), I32),
            jax.ShapeDtypeStruct((PTOT + 128,), I32),
        ],
        mesh=_MESH,
        scratch_types=[
            pltpu.VMEM((CH,), I32),
            pltpu.VMEM((CH,), I32),
            pltpu.VMEM((CH,), I32),
            pltpu.VMEM((CH,), I32),
            pltpu.SemaphoreType.DMA,
        ],
    )()


def _accum2_kernel(n, np_):
    RT = np_ // NW
    RTA = RT + 8

    def body(h_ref, rs_ref, rd_ref, nch_ref, out_ref, rsb, rdb, rows0, rows1,
             acc, cv, sem0, sem1):
        w = _wid()
        _zero_rows(acc, RTA)

        pltpu.sync_copy(nch_ref.at[pl.ds(w * 16, 16)], cv)
        nch = cv[pl.ds(0, 16)][0]
        pltpu.sync_copy(rs_ref.at[pl.ds(w * PT, PT)], rsb)
        pltpu.sync_copy(rd_ref.at[pl.ds(w * PT, PT)], rdb)

        def gather(c, rows, sem):
            return pltpu.make_async_copy(
                h_ref.at[rsb.at[pl.ds(c * CH, CH)]], rows, sem)

        @pl.when(nch > 0)
        def _():
            gather(0, rows0, sem0).start()

        def process(c, rows):
            for j in range(CH // 16):
                dv = rdb[pl.ds(c * CH + j * 16, 16)]
                for l in range(16):
                    r = dv[l]
                    e = j * 16 + l
                    for q in range(NHID // 16):
                        sl = pl.ds(q * 16, 16)
                        acc[r, sl] = acc[r, sl] + rows[e, sl]

        def outer(k, _):
            c0 = 2 * k

            @pl.when(c0 < nch)
            def _():
                @pl.when(c0 + 1 < nch)
                def _():
                    gather(c0 + 1, rows1, sem1).start()
                gather(c0, rows0, sem0).wait()
                process(c0, rows0)
            c1 = c0 + 1

            @pl.when(c1 < nch)
            def _():
                @pl.when(c1 + 1 < nch)
                def _():
                    gather(c1 + 1, rows0, sem0).start()
                gather(c1, rows1, sem1).wait()
                process(c1, rows1)
            return 0
        lax.fori_loop(0, (nch + 1) // 2, outer, 0)

        pltpu.sync_copy(acc.at[pl.ds(0, RT)], out_ref.at[pl.ds(w * RT, RT)])

    return functools.partial(
        pl.kernel, body,
        out_type=jax.ShapeDtypeStruct((np_, NHID), F32),
        mesh=_MESH,
        scratch_types=[
            pltpu.VMEM((PT,), I32),
            pltpu.VMEM((PT,), I32),
            pltpu.VMEM((CH, NHID), F32),
            pltpu.VMEM((CH, NHID), F32),
            pltpu.VMEM((RTA, NHID), F32),
            pltpu.VMEM((16,), I32),
            pltpu.SemaphoreType.DMA,
            pltpu.SemaphoreType.DMA,
        ],
    )()


def _build_kernel(np_prev, ps):
    per_sub = ps // 16
    zcount = np_prev // 16

    def body(perm_ref, enc_ref, idxb, valb, sem):
        c = lax.axis_index("c")
        s = lax.axis_index("s")
        iota = lax.iota(I32, 16)

        @pl.when(c == 0)
        def _():
            neg = jnp.full((16,), -1, I32)
            for j in range(CH // 16):
                valb[pl.ds(j * 16, 16)] = neg
            _fill_copy(valb, enc_ref, s * zcount, zcount)
            plsc.subcore_barrier()

            nchunk = per_sub // CH

            def sc(ci, _):
                base = s * per_sub + ci * CH
                pltpu.sync_copy(perm_ref.at[pl.ds(base, CH)], idxb)
                for j in range(CH // 16):
                    valb[pl.ds(j * 16, 16)] = base + j * 16 + iota
                pltpu.sync_copy(valb, enc_ref.at[idxb])
                return 0
            lax.fori_loop(0, nchunk, sc, 0)

    return functools.partial(
        pl.kernel, body,
        out_type=jax.ShapeDtypeStruct((np_prev,), I32),
        mesh=_MESH,
        scratch_types=[
            pltpu.VMEM((CH,), I32),
            pltpu.VMEM((CH,), I32),
            pltpu.SemaphoreType.DMA,
        ],
    )()


def _preprocess_kernel(n_new):
    def body(src_ref, dst_ref, enc_ref, srcn_ref, dstn_ref,
             sb, db, ns, nd, so, do, sem, sem2):
        w = _wid()
        iota = lax.iota(I32, 16)
        base = w * EPW

        def chunk(ci, _):
            off = base + ci * CH
            pltpu.sync_copy(src_ref.at[pl.ds(off, CH)], sb)
            pltpu.sync_copy(dst_ref.at[pl.ds(off, CH)], db)
            cp1 = pltpu.async_copy(enc_ref.at[sb], ns, sem)
            cp2 = pltpu.async_copy(enc_ref.at[db], nd, sem2)
            cp1.wait()
            cp2.wait()
            for j in range(CH // 16):
                sl = pl.ds(j * 16, 16)
                nsv = ns[sl]
                ndv = nd[sl]
                valid = (nsv >= 0) & (ndv >= 0)
                spread = (j % 4) * 16 + iota
                so[sl] = jnp.where(valid, nsv, spread)
                do[sl] = jnp.where(valid, ndv, n_new + spread)
            pltpu.sync_copy(so, srcn_ref.at[pl.ds(off, CH)])
            pltpu.sync_copy(do, dstn_ref.at[pl.ds(off, CH)])
            return 0
        lax.fori_loop(0, NCHUNK, chunk, 0)

    return functools.partial(
        pl.kernel, body,
        out_type=[
            jax.ShapeDtypeStruct((E,), I32),
            jax.ShapeDtypeStruct((E,), I32),
        ],
        mesh=_MESH,
        scratch_types=[pltpu.VMEM((CH,), I32) for _ in range(6)]
        + [pltpu.SemaphoreType.DMA, pltpu.SemaphoreType.DMA],
    )()


def _gate_kernel(n, k, cg, ps):
    per_w = ps // NW

    def body(x_ref, perm_ref, out_ref, idxb, rows, sem):
        w = _wid()
        nchunk = per_w // cg

        def chunk(ci, _):
            base = w * per_w + ci * cg
            pltpu.sync_copy(perm_ref.at[pl.ds(base, cg)], idxb)
            pltpu.async_copy(x_ref.at[idxb], rows, sem).wait()
            pltpu.sync_copy(rows, out_ref.at[pl.ds(base, cg)])
            return 0
        lax.fori_loop(0, nchunk, chunk, 0)

    return functools.partial(
        pl.kernel, body,
        out_type=jax.ShapeDtypeStruct((ps, NHID), F32),
        mesh=_MESH,
        scratch_types=[
            pltpu.VMEM((cg,), I32),
            pltpu.VMEM((cg, NHID), F32),
            pltpu.SemaphoreType.DMA,
        ],
    )()


def _gatemul_body(x_ref, g_ref, out_ref):
    out_ref[...] = x_ref[...] * g_ref[...]


def _gatemul(xg, gate, k):
    blk = 256
    grid = (k + blk - 1) // blk
    return pl.pallas_call(
        _gatemul_body,
        grid=(grid,),
        in_specs=[
            pl.BlockSpec((blk, NHID), lambda i: (i, 0)),
            pl.BlockSpec((blk, 1), lambda i: (i, 0)),
        ],
        out_specs=pl.BlockSpec((blk, NHID), lambda i: (i, 0)),
        out_shape=jax.ShapeDtypeStruct((k, NHID), F32),
    )(xg, gate.reshape(k, 1))


def _conv(h0, acc, srcd, dstd, fw, n, np_):
    RT = np_ // NW
    PTOT = NW * PT
    valid = dstd < n
    owner = jnp.where(valid, dstd, 0) // RT
    oh = ((owner[:, None] == jnp.arange(NW, dtype=I32)[None, :])
          & valid[:, None]).astype(I32).reshape(E // 128, 128, NW)
    within = jnp.cumsum(oh, axis=1)
    blk_counts = within[:, -1, :]
    base_blocks = jnp.cumsum(blk_counts, axis=0) - blk_counts
    pos3 = base_blocks[:, None, :] + within
    pos = jnp.sum(pos3 * oh, axis=2).reshape(E) - 1
    counts = blk_counts.sum(axis=0)
    addr = jnp.where(valid, owner * PT + pos,
                     PTOT + (jnp.arange(E, dtype=I32) % 64))
    dl = jnp.where(valid, dstd - owner * RT, 0)
    nch = (jnp.minimum(counts, PT - CH) + (CH - 1)) // CH
    nchv = jnp.broadcast_to(nch[:, None], (NW, 16)).reshape(NW * 16)
    pada = (jnp.arange(NW, dtype=I32)[:, None] * PT
            + jnp.minimum(counts, PT - CH)[:, None]
            + jnp.arange(CH, dtype=I32)[None, :]).reshape(NW * CH)

    routed_s, routed_d = _place_kernel(n, np_)(srcd, dl, addr, pada)
    accum = _accum2_kernel(n, np_)
    h = h0
    for i in range(1, int(fw.shape[0])):
        hp = accum(h, routed_s, routed_d, nchv)
        h, acc = _axpy(acc, hp[:n], fw[i].reshape(1, 1), n)
    return acc


def _gate_pads(k, cg):
    span32 = (k + NW - 1) // NW
    per_w = (span32 + cg - 1) // cg * cg
    return per_w * NW


def _build_pads(k):
    span16 = (k + 15) // 16
    per_sub = (span16 + CH - 1) // CH * CH
    return per_sub * 16


def _pool(x, srcd, dstd, p, n, np_, k, cg, last):
    score = _score(x, p)
    topv, perm = lax.top_k(score, k)
    gate = jnp.tanh(topv)

    psg = _gate_pads(k, cg)
    pad_g = psg - k
    perm_g = jnp.concatenate(
        [perm, (jnp.arange(pad_g, dtype=I32) % 64)])
    xg = _gate_kernel(n, k, cg, psg)(x, perm_g)[:k]
    xk = _gatemul(xg, gate, k)

    if last:
        return xk, None, None, perm

    psb = _build_pads(k)
    pad_b = psb - k
    free = np_ - (n + 64)
    perm_b = jnp.concatenate(
        [perm, n + 64 + (jnp.arange(pad_b, dtype=I32) % free)])
    enc = _build_kernel(np_, psb)(perm_b)
    srcn, dstn = _preprocess_kernel(k)(srcd, dstd, enc)
    return xk, srcn, dstn, perm


def kernel(x, edge_index, edge_attr, batch, W_lump, b_lump, W1, b1, fw1, p1,
           W2, b2, fw2, p2, W3, b3, fw3, p3, Wl1, bl1, Wl2, bl2):
    del edge_attr, batch
    src = edge_index[0]
    dst = edge_index[1]

    N1, N2, N3, N4 = 10000, 5000, 2500, 1250
    NP1, NP2, NP3 = _pad16(N1), _pad16(N2), _pad16(N3)

    dst_e1 = _pre1(src, dst)
    src_e1 = src

    x0, _ = _lin(x, W_lump, b_lump, jnp.zeros((1, 1), F32))
    h0, acc = _lin(x0, W1, b1, fw1[0].reshape(1, 1))
    x1 = _conv(h0, acc, src_e1, dst_e1, fw1, N1, NP1)
    xk1, src_e2, dst_e2, perm1 = _pool(x1, src_e1, dst_e1, p1, N1, NP1, N2,
                                       CH, False)

    h0, acc = _lin(xk1, W2, b2, fw2[0].reshape(1, 1))
    x2 = _conv(h0, acc, src_e2, dst_e2, fw2, N2, NP2)
    xk2, src_e3, dst_e3, perm2 = _pool(x2, src_e2, dst_e2, p2, N2, NP2, N3,
                                       CH, False)

    h0, acc = _lin(xk2, W3, b3, fw3[0].reshape(1, 1))
    x3 = _conv(h0, acc, src_e3, dst_e3, fw3, N3, NP3)
    xk3, _, _, perm3 = _pool(x3, None, None, p3, N3, NP3, N4, 40, True)

    out = _mlp(xk3, Wl1, bl1, Wl2, bl2)
    return (out, perm1, perm2, perm3)
